# Initial kernel scaffold; baseline (speedup 1.0000x reference)
import jax
import jax.numpy as jnp
from jax.experimental import pallas as pl


def kernel(x, edge_index, edge_attr, u, batch, W1, b1, gamma, beta, W2, b2):
    raise NotImplementedError("write your pallas kernel here")


# speedup vs baseline: 4.9529x; 4.9529x over previous
import functools

import jax
import jax.numpy as jnp
from jax import lax
from jax.experimental import pallas as pl
from jax.experimental.pallas import tpu as pltpu
from jax.experimental.pallas import tpu_sc as plsc

_NC = 2
_NS = 16
_NW = _NC * _NS

_B = 80
_KI = 5


def _sc_scatter_partials(idx2d, attr, zeros):
    npad, de = zeros.shape
    e = attr.shape[0]
    epw = e // _NW
    rows_pw = epw // _B
    nchunk = rows_pw // _KI
    chunk = _B * _KI
    rps = npad // _NS

    mesh = plsc.VectorSubcoreMesh(core_axis_name="c", subcore_axis_name="s")

    @functools.partial(
        pl.kernel,
        mesh=mesh,
        out_type=jax.ShapeDtypeStruct((_NC, npad, de), jnp.float32),
        scratch_types=[
            pltpu.VMEM((_KI, _B), jnp.int32),
            pltpu.VMEM((chunk, de), jnp.float32),
            pltpu.VMEM_SHARED((npad, de), jnp.float32),
        ],
        compiler_params=pltpu.CompilerParams(use_tc_tiling_on_sc=False),
    )
    def k(idx_hbm, attr_hbm, z_hbm, out_hbm, idx_v, attr_v, agg_sh):
        cid = lax.axis_index("c")
        sid = lax.axis_index("s")
        wid = sid * _NC + cid

        pltpu.sync_copy(z_hbm.at[pl.ds(sid * rps, rps)],
                        agg_sh.at[pl.ds(sid * rps, rps)])
        plsc.subcore_barrier()

        row0 = wid * rows_pw
        e0 = wid * epw

        @pl.loop(0, nchunk)
        def _(c):
            pltpu.sync_copy(idx_hbm.at[pl.ds(row0 + c * _KI, _KI)], idx_v)
            pltpu.sync_copy(attr_hbm.at[pl.ds(e0 + c * chunk, chunk)], attr_v)
            for j in range(_KI):
                pltpu.sync_copy(attr_v.at[pl.ds(j * _B, _B)],
                                agg_sh.at[idx_v.at[j]], add=True)

        plsc.subcore_barrier()
        pltpu.sync_copy(agg_sh.at[pl.ds(sid * rps, rps)],
                        out_hbm.at[cid, pl.ds(sid * rps, rps)])

    return k(idx2d, attr, zeros)


_BN = 1000


def _tc_stats(x, parts, w1x, w1a, b1):
    n, f = x.shape
    de = parts.shape[2]
    h = w1x.shape[1]
    grid = n // _BN

    def body(x_ref, p_ref, wx_ref, wa_ref, b_ref, s1_ref, s2_ref):
        agg = p_ref[0] + p_ref[1]
        hv = jnp.dot(x_ref[...], wx_ref[...], preferred_element_type=jnp.float32)
        hv += jnp.dot(agg, wa_ref[...], preferred_element_type=jnp.float32)
        hv = jnp.maximum(hv + b_ref[...], 0.0)

        @pl.when(pl.program_id(0) == 0)
        def _():
            s1_ref[...] = jnp.zeros_like(s1_ref)
            s2_ref[...] = jnp.zeros_like(s2_ref)

        s1_ref[...] += jnp.sum(hv, axis=0, keepdims=True)
        s2_ref[...] += jnp.sum(hv * hv, axis=0, keepdims=True)

    return pl.pallas_call(
        body,
        grid=(grid,),
        in_specs=[
            pl.BlockSpec((_BN, f), lambda i: (i, 0)),
            pl.BlockSpec((_NC, _BN, de), lambda i: (0, i, 0)),
            pl.BlockSpec((f, h), lambda i: (0, 0)),
            pl.BlockSpec((de, h), lambda i: (0, 0)),
            pl.BlockSpec((1, h), lambda i: (0, 0)),
        ],
        out_specs=[
            pl.BlockSpec((1, h), lambda i: (0, 0)),
            pl.BlockSpec((1, h), lambda i: (0, 0)),
        ],
        out_shape=[jax.ShapeDtypeStruct((1, h), jnp.float32)] * 2,
    )(x, parts, w1x, w1a, b1)


def _tc_out(x, parts, w1x, w1a, b1, s1, s2, gamma, beta, w2, b2):
    n, f = x.shape
    de = parts.shape[2]
    h = w1x.shape[1]
    grid = n // _BN
    inv_n = 1.0 / n

    def body(x_ref, p_ref, wx_ref, wa_ref, b_ref, s1_ref, s2_ref,
             g_ref, be_ref, w2_ref, b2_ref, o_ref):
        agg = p_ref[0] + p_ref[1]
        hv = jnp.dot(x_ref[...], wx_ref[...], preferred_element_type=jnp.float32)
        hv += jnp.dot(agg, wa_ref[...], preferred_element_type=jnp.float32)
        hv = jnp.maximum(hv + b_ref[...], 0.0)

        mean = s1_ref[...] * inv_n
        var = s2_ref[...] * inv_n - mean * mean
        scale = g_ref[...] * lax.rsqrt(var + 1e-5)
        shift = be_ref[...] - mean * scale
        hn = hv * scale + shift
        o_ref[...] = jnp.dot(hn, w2_ref[...],
                             preferred_element_type=jnp.float32) + b2_ref[...]

    return pl.pallas_call(
        body,
        grid=(grid,),
        in_specs=[
            pl.BlockSpec((_BN, f), lambda i: (i, 0)),
            pl.BlockSpec((_NC, _BN, de), lambda i: (0, i, 0)),
            pl.BlockSpec((f, h), lambda i: (0, 0)),
            pl.BlockSpec((de, h), lambda i: (0, 0)),
            pl.BlockSpec((1, h), lambda i: (0, 0)),
            pl.BlockSpec((1, h), lambda i: (0, 0)),
            pl.BlockSpec((1, h), lambda i: (0, 0)),
            pl.BlockSpec((1, h), lambda i: (0, 0)),
            pl.BlockSpec((1, h), lambda i: (0, 0)),
            pl.BlockSpec((h, f), lambda i: (0, 0)),
            pl.BlockSpec((1, f), lambda i: (0, 0)),
        ],
        out_specs=pl.BlockSpec((_BN, f), lambda i: (i, 0)),
        out_shape=jax.ShapeDtypeStruct((n, f), jnp.float32),
    )(x, parts, w1x, w1a, b1, s1, s2, gamma, beta, w2, b2)


def kernel(x, edge_index, edge_attr, u, batch, W1, b1, gamma, beta, W2, b2):
    n, f = x.shape
    e, de = edge_attr.shape
    h = W1.shape[1]

    npad = ((n + 127) // 128) * 128

    row = edge_index[0]
    idx2d = row.reshape(e // _B, _B)
    zeros = jnp.zeros((npad, de), jnp.float32)

    parts = _sc_scatter_partials(idx2d, edge_attr, zeros)

    w1x = W1[:f]
    w1a = W1[f:]
    s1, s2 = _tc_stats(x, parts, w1x, w1a, b1.reshape(1, h))
    return _tc_out(x, parts, w1x, w1a, b1.reshape(1, h), s1, s2,
                   gamma.reshape(1, h), beta.reshape(1, h),
                   W2, b2.reshape(1, f))


# speedup vs baseline: 5.4481x; 1.1000x over previous
import functools

import jax
import jax.numpy as jnp
from jax import lax
from jax.experimental import pallas as pl
from jax.experimental.pallas import tpu as pltpu
from jax.experimental.pallas import tpu_sc as plsc

_NC = 2
_NS = 16
_NW = _NC * _NS

_B = 80
_KI = 8
_LW = 128


def _sc_scatter_partials(idx2d, attr, zeros, e, de, npad):
    chunk = _B * _KI
    nchunks = e // chunk
    base_trips = nchunks // _NW
    rem = nchunks % _NW
    rps = npad // _NS

    mesh = plsc.VectorSubcoreMesh(core_axis_name="c", subcore_axis_name="s")

    @functools.partial(
        pl.kernel,
        mesh=mesh,
        out_type=jax.ShapeDtypeStruct((_NC, npad, de), jnp.float32),
        scratch_types=[
            pltpu.VMEM((_KI, _B), jnp.int32),
            pltpu.VMEM((chunk, de), jnp.float32),
            pltpu.VMEM_SHARED((npad, de), jnp.float32),
        ],
        compiler_params=pltpu.CompilerParams(use_tc_tiling_on_sc=False),
    )
    def k(idx_hbm, attr_hbm, z_hbm, out_hbm, idx_v, attr_v, agg_sh):
        cid = lax.axis_index("c")
        sid = lax.axis_index("s")
        wid = sid * _NC + cid

        pltpu.sync_copy(z_hbm.at[pl.ds(sid * rps, rps)],
                        agg_sh.at[pl.ds(sid * rps, rps)])
        plsc.subcore_barrier()

        def do_chunk(g):
            pltpu.sync_copy(idx_hbm.at[pl.ds(g * _KI, _KI)], idx_v)
            pltpu.sync_copy(attr_hbm.at[pl.ds(g * chunk, chunk)], attr_v)
            for j in range(_KI):
                pltpu.sync_copy(attr_v.at[pl.ds(j * _B, _B)],
                                agg_sh.at[idx_v.at[j]], add=True)

        @pl.loop(0, base_trips)
        def _(c):
            do_chunk(c * _NW + wid)

        if rem:
            @pl.when(wid < rem)
            def _():
                do_chunk(base_trips * _NW + wid)

        plsc.subcore_barrier()
        pltpu.sync_copy(agg_sh.at[pl.ds(sid * rps, rps)],
                        out_hbm.at[cid, pl.ds(sid * rps, rps)])

    return k(idx2d, attr, zeros)


_BE = 12800


def _tc_attr_relayout(attr_t):
    de, e = attr_t.shape
    grid = e // _BE

    def body(x_ref, o_ref):
        g = _LW // de
        y = x_ref[...].T.reshape(_BE // g, g, de)
        o_ref[...] = jnp.concatenate([y[:, a, :] for a in range(g)], axis=1)

    return pl.pallas_call(
        body,
        grid=(grid,),
        in_specs=[pl.BlockSpec((de, _BE), lambda i: (0, i))],
        out_specs=pl.BlockSpec((_BE * de // _LW, _LW), lambda i: (i, 0)),
        out_shape=jax.ShapeDtypeStruct((e * de // _LW, _LW), jnp.float32),
    )(attr_t)


_BN = 1000


def _tc_stats(x, parts, w1x, w1a, b1):
    n, f = x.shape
    de = parts.shape[2]
    h = w1x.shape[1]
    grid = n // _BN

    def body(x_ref, p_ref, wx_ref, wa_ref, b_ref, s1_ref, s2_ref):
        agg = p_ref[0] + p_ref[1]
        hv = jnp.dot(x_ref[...], wx_ref[...], preferred_element_type=jnp.float32)
        hv += jnp.dot(agg, wa_ref[...], preferred_element_type=jnp.float32)
        hv = jnp.maximum(hv + b_ref[...], 0.0)

        @pl.when(pl.program_id(0) == 0)
        def _():
            s1_ref[...] = jnp.zeros_like(s1_ref)
            s2_ref[...] = jnp.zeros_like(s2_ref)

        s1_ref[...] += jnp.sum(hv, axis=0, keepdims=True)
        s2_ref[...] += jnp.sum(hv * hv, axis=0, keepdims=True)

    return pl.pallas_call(
        body,
        grid=(grid,),
        in_specs=[
            pl.BlockSpec((_BN, f), lambda i: (i, 0)),
            pl.BlockSpec((_NC, _BN, de), lambda i: (0, i, 0)),
            pl.BlockSpec((f, h), lambda i: (0, 0)),
            pl.BlockSpec((de, h), lambda i: (0, 0)),
            pl.BlockSpec((1, h), lambda i: (0, 0)),
        ],
        out_specs=[
            pl.BlockSpec((1, h), lambda i: (0, 0)),
            pl.BlockSpec((1, h), lambda i: (0, 0)),
        ],
        out_shape=[jax.ShapeDtypeStruct((1, h), jnp.float32)] * 2,
    )(x, parts, w1x, w1a, b1)


def _tc_out(x, parts, w1x, w1a, b1, s1, s2, gamma, beta, w2, b2):
    n, f = x.shape
    de = parts.shape[2]
    h = w1x.shape[1]
    grid = n // _BN
    inv_n = 1.0 / n

    def body(x_ref, p_ref, wx_ref, wa_ref, b_ref, s1_ref, s2_ref,
             g_ref, be_ref, w2_ref, b2_ref, o_ref):
        agg = p_ref[0] + p_ref[1]
        hv = jnp.dot(x_ref[...], wx_ref[...], preferred_element_type=jnp.float32)
        hv += jnp.dot(agg, wa_ref[...], preferred_element_type=jnp.float32)
        hv = jnp.maximum(hv + b_ref[...], 0.0)

        mean = s1_ref[...] * inv_n
        var = s2_ref[...] * inv_n - mean * mean
        scale = g_ref[...] * lax.rsqrt(var + 1e-5)
        shift = be_ref[...] - mean * scale
        hn = hv * scale + shift
        o_ref[...] = jnp.dot(hn, w2_ref[...],
                             preferred_element_type=jnp.float32) + b2_ref[...]

    return pl.pallas_call(
        body,
        grid=(grid,),
        in_specs=[
            pl.BlockSpec((_BN, f), lambda i: (i, 0)),
            pl.BlockSpec((_NC, _BN, de), lambda i: (0, i, 0)),
            pl.BlockSpec((f, h), lambda i: (0, 0)),
            pl.BlockSpec((de, h), lambda i: (0, 0)),
            pl.BlockSpec((1, h), lambda i: (0, 0)),
            pl.BlockSpec((1, h), lambda i: (0, 0)),
            pl.BlockSpec((1, h), lambda i: (0, 0)),
            pl.BlockSpec((1, h), lambda i: (0, 0)),
            pl.BlockSpec((1, h), lambda i: (0, 0)),
            pl.BlockSpec((h, f), lambda i: (0, 0)),
            pl.BlockSpec((1, f), lambda i: (0, 0)),
        ],
        out_specs=pl.BlockSpec((_BN, f), lambda i: (i, 0)),
        out_shape=jax.ShapeDtypeStruct((n, f), jnp.float32),
    )(x, parts, w1x, w1a, b1, s1, s2, gamma, beta, w2, b2)


def kernel(x, edge_index, edge_attr, u, batch, W1, b1, gamma, beta, W2, b2):
    n, f = x.shape
    e, de = edge_attr.shape
    h = W1.shape[1]

    npad = ((n + 127) // 128) * 128

    idx2d = edge_index[0].reshape(e // _B, _B)
    attr_lin = _tc_attr_relayout(edge_attr.T).reshape(e, de)
    zeros = jnp.zeros((npad, de), jnp.float32)

    parts = _sc_scatter_partials(idx2d, attr_lin, zeros, e, de, npad)

    w1x = W1[:f]
    w1a = W1[f:]
    s1, s2 = _tc_stats(x, parts, w1x, w1a, b1.reshape(1, h))
    return _tc_out(x, parts, w1x, w1a, b1.reshape(1, h), s1, s2,
                   gamma.reshape(1, h), beta.reshape(1, h),
                   W2, b2.reshape(1, f))
